# Initial kernel scaffold; baseline (speedup 1.0000x reference)
import jax
import jax.numpy as jnp
from jax.experimental import pallas as pl


def kernel(x, Wd, bd, Wu, bu):
    raise NotImplementedError("write your pallas kernel here")


# speedup vs baseline: 14.0016x; 14.0016x over previous
import functools

import jax
import jax.numpy as jnp
import numpy as np
from jax.experimental import pallas as pl
from jax.experimental.pallas import tpu as pltpu

_ND = 12
_NC = 1 << _ND
_LAT = 1024
_T = 256

_MB = (4.0 * ((np.arange(_NC)[None, :] >> np.arange(_ND)[:, None]) & 1)
       * (2.0 ** np.arange(_ND))[:, None]).astype(np.float32)


def _lfq_body(x_ref, wd_ref, bd_ref, wu_ref, bu_ref, mb_ref,
              tok_ref, zup_ref, zqup_ref, lp_ref, closs_ref):
    x = x_ref[...]
    u = jnp.dot(x, wd_ref[...], preferred_element_type=jnp.float32)
    z = jnp.tanh(u + bd_ref[...])

    bits = z > 0.0
    zq = jnp.where(bits, 1.0, -1.0).astype(jnp.float32)

    zup_ref[...] = (jnp.dot(z, wu_ref[...], preferred_element_type=jnp.float32)
                    + bu_ref[...])
    zqup_ref[...] = (jnp.dot(zq, wu_ref[...], preferred_element_type=jnp.float32)
                     + bu_ref[...])

    ii = jax.lax.broadcasted_iota(jnp.int32, (1, _ND), 1)
    pow2 = (1 << ii).astype(jnp.float32)
    tokf = jnp.sum(jnp.where(bits, pow2, 0.0), axis=1, keepdims=True)
    tok_ref[...] = tokf.astype(jnp.int32)

    d = z - zq
    part = jnp.sum(jnp.sum(d * d, axis=1, keepdims=True), axis=0,
                   keepdims=True)
    closs_ref[...] = part.reshape(1, 1, 1)

    logits = jnp.dot(z, mb_ref[...], preferred_element_type=jnp.float32)
    c = 4.0 * z * pow2
    sp = jnp.maximum(c, 0.0) + jnp.log1p(jnp.exp(-jnp.abs(c)))
    lse = jnp.sum(sp, axis=1, keepdims=True)
    lp_ref[...] = logits - lse


@functools.partial(jax.jit, static_argnames=("interpret",))
def kernel(x, Wd, bd, Wu, bu, *, interpret=False):
    B, S, LAT = x.shape
    n_tok = B * S
    grid = (n_tok // _T,)

    x2 = x.reshape(n_tok, LAT)
    bd2 = bd.reshape(1, _ND)
    bu2 = bu.reshape(1, LAT)
    mb = jnp.asarray(_MB)

    out_shapes = (
        jax.ShapeDtypeStruct((n_tok, 1), jnp.int32),
        jax.ShapeDtypeStruct((n_tok, LAT), jnp.float32),
        jax.ShapeDtypeStruct((n_tok, LAT), jnp.float32),
        jax.ShapeDtypeStruct((n_tok, _NC), jnp.float32),
        jax.ShapeDtypeStruct((grid[0], 1, 1), jnp.float32),
    )
    out_specs = (
        pl.BlockSpec((_T, 1), lambda i: (i, 0)),
        pl.BlockSpec((_T, LAT), lambda i: (i, 0)),
        pl.BlockSpec((_T, LAT), lambda i: (i, 0)),
        pl.BlockSpec((_T, _NC), lambda i: (i, 0)),
        pl.BlockSpec((1, 1, 1), lambda i: (i, 0, 0)),
    )
    in_specs = [
        pl.BlockSpec((_T, LAT), lambda i: (i, 0)),
        pl.BlockSpec((LAT, _ND), lambda i: (0, 0)),
        pl.BlockSpec((1, _ND), lambda i: (0, 0)),
        pl.BlockSpec((_ND, LAT), lambda i: (0, 0)),
        pl.BlockSpec((1, LAT), lambda i: (0, 0)),
        pl.BlockSpec((_ND, _NC), lambda i: (0, 0)),
    ]

    tok, zup, zqup, lp, closs_part = pl.pallas_call(
        _lfq_body,
        out_shape=out_shapes,
        grid=grid,
        in_specs=in_specs,
        out_specs=out_specs,
        compiler_params=pltpu.CompilerParams(
            dimension_semantics=("parallel",),
        ),
        name="lfq_fused",
        interpret=interpret,
    )(x2, Wd, bd2, Wu, bu2, mb)

    tokens = tok.reshape(B, S)
    z_up = zup.reshape(B, S, LAT)
    z_q_up = zqup.reshape(B, S, LAT)
    token_log_probs = lp.reshape(B, S, _NC)
    commit_loss = jnp.sum(closs_part) / (n_tok * _ND)
    return tokens, z_up, z_q_up, token_log_probs, commit_loss


# speedup vs baseline: 14.4945x; 1.0352x over previous
import functools

import jax
import jax.numpy as jnp
import numpy as np
from jax.experimental import pallas as pl
from jax.experimental.pallas import tpu as pltpu

_ND = 12
_NC = 1 << _ND
_LAT = 1024
_T = 512

_MB = (4.0 * ((np.arange(_NC)[None, :] >> np.arange(_ND)[:, None]) & 1)
       * (2.0 ** np.arange(_ND))[:, None]).astype(np.float32)


def _lfq_body(x_ref, wd_ref, bd_ref, wu_ref, bu_ref, mb_ref,
              tok_ref, zup_ref, zqup_ref, lp_ref, closs_ref):
    x = x_ref[...]
    u = jnp.dot(x, wd_ref[...], preferred_element_type=jnp.float32)
    z = jnp.tanh(u + bd_ref[...])

    bits = z > 0.0
    zq = jnp.where(bits, 1.0, -1.0).astype(jnp.float32)

    zup_ref[...] = (jnp.dot(z, wu_ref[...], preferred_element_type=jnp.float32)
                    + bu_ref[...])
    zqup_ref[...] = (jnp.dot(zq, wu_ref[...], preferred_element_type=jnp.float32)
                     + bu_ref[...])

    ii = jax.lax.broadcasted_iota(jnp.int32, (1, _ND), 1)
    pow2 = (1 << ii).astype(jnp.float32)
    tokf = jnp.sum(jnp.where(bits, pow2, 0.0), axis=1, keepdims=True)
    tok_ref[...] = tokf.astype(jnp.int32)

    d = z - zq
    part = jnp.sum(jnp.sum(d * d, axis=1, keepdims=True), axis=0,
                   keepdims=True)
    closs_ref[...] = part.reshape(1, 1, 1)

    logits = jnp.dot(z, mb_ref[...], preferred_element_type=jnp.float32)
    c = 4.0 * z * pow2
    sp = jnp.maximum(c, 0.0) + jnp.log1p(jnp.exp(-jnp.abs(c)))
    lse = jnp.sum(sp, axis=1, keepdims=True)
    lp_ref[...] = logits - lse


@functools.partial(jax.jit, static_argnames=("interpret",))
def kernel(x, Wd, bd, Wu, bu, *, interpret=False):
    B, S, LAT = x.shape
    n_tok = B * S
    grid = (n_tok // _T,)

    x2 = x.reshape(n_tok, LAT)
    bd2 = bd.reshape(1, _ND)
    bu2 = bu.reshape(1, LAT)
    mb = jnp.asarray(_MB)

    out_shapes = (
        jax.ShapeDtypeStruct((n_tok, 1), jnp.int32),
        jax.ShapeDtypeStruct((n_tok, LAT), jnp.float32),
        jax.ShapeDtypeStruct((n_tok, LAT), jnp.float32),
        jax.ShapeDtypeStruct((n_tok, _NC), jnp.float32),
        jax.ShapeDtypeStruct((grid[0], 1, 1), jnp.float32),
    )
    out_specs = (
        pl.BlockSpec((_T, 1), lambda i: (i, 0)),
        pl.BlockSpec((_T, LAT), lambda i: (i, 0)),
        pl.BlockSpec((_T, LAT), lambda i: (i, 0)),
        pl.BlockSpec((_T, _NC), lambda i: (i, 0)),
        pl.BlockSpec((1, 1, 1), lambda i: (i, 0, 0)),
    )
    in_specs = [
        pl.BlockSpec((_T, LAT), lambda i: (i, 0)),
        pl.BlockSpec((LAT, _ND), lambda i: (0, 0)),
        pl.BlockSpec((1, _ND), lambda i: (0, 0)),
        pl.BlockSpec((_ND, LAT), lambda i: (0, 0)),
        pl.BlockSpec((1, LAT), lambda i: (0, 0)),
        pl.BlockSpec((_ND, _NC), lambda i: (0, 0)),
    ]

    tok, zup, zqup, lp, closs_part = pl.pallas_call(
        _lfq_body,
        out_shape=out_shapes,
        grid=grid,
        in_specs=in_specs,
        out_specs=out_specs,
        compiler_params=pltpu.CompilerParams(
            dimension_semantics=("parallel",),
        ),
        name="lfq_fused",
        interpret=interpret,
    )(x2, Wd, bd2, Wu, bu2, mb)

    tokens = tok.reshape(B, S)
    z_up = zup.reshape(B, S, LAT)
    z_q_up = zqup.reshape(B, S, LAT)
    token_log_probs = lp.reshape(B, S, _NC)
    commit_loss = jnp.sum(closs_part) / (n_tok * _ND)
    return tokens, z_up, z_q_up, token_log_probs, commit_loss


# speedup vs baseline: 14.6296x; 1.0093x over previous
import functools

import jax
import jax.numpy as jnp
import numpy as np
from jax.experimental import pallas as pl
from jax.experimental.pallas import tpu as pltpu

_ND = 12
_NC = 1 << _ND
_LAT = 1024
_T = 1024

_MB = (4.0 * ((np.arange(_NC)[None, :] >> np.arange(_ND)[:, None]) & 1)
       * (2.0 ** np.arange(_ND))[:, None]).astype(np.float32)


def _lfq_body(x_ref, wd_ref, bd_ref, wu_ref, bu_ref, mb_ref,
              tok_ref, zup_ref, zqup_ref, lp_ref, closs_ref):
    x = x_ref[...]
    u = jnp.dot(x, wd_ref[...], preferred_element_type=jnp.float32)
    z = jnp.tanh(u + bd_ref[...])

    bits = z > 0.0
    zq = jnp.where(bits, 1.0, -1.0).astype(jnp.float32)

    zup_ref[...] = (jnp.dot(z, wu_ref[...], preferred_element_type=jnp.float32)
                    + bu_ref[...])
    zqup_ref[...] = (jnp.dot(zq, wu_ref[...], preferred_element_type=jnp.float32)
                     + bu_ref[...])

    ii = jax.lax.broadcasted_iota(jnp.int32, (1, _ND), 1)
    pow2 = (1 << ii).astype(jnp.float32)
    tokf = jnp.sum(jnp.where(bits, pow2, 0.0), axis=1, keepdims=True)
    tok_ref[...] = tokf.astype(jnp.int32)

    d = z - zq
    part = jnp.sum(jnp.sum(d * d, axis=1, keepdims=True), axis=0,
                   keepdims=True)
    closs_ref[...] = part.reshape(1, 1, 1)

    logits = jnp.dot(z, mb_ref[...], preferred_element_type=jnp.float32)
    c = 4.0 * z * pow2
    sp = jnp.maximum(c, 0.0) + jnp.log1p(jnp.exp(-jnp.abs(c)))
    lse = jnp.sum(sp, axis=1, keepdims=True)
    lp_ref[...] = logits - lse


@functools.partial(jax.jit, static_argnames=("interpret",))
def kernel(x, Wd, bd, Wu, bu, *, interpret=False):
    B, S, LAT = x.shape
    n_tok = B * S
    grid = (n_tok // _T,)

    x2 = x.reshape(n_tok, LAT)
    bd2 = bd.reshape(1, _ND)
    bu2 = bu.reshape(1, LAT)
    mb = jnp.asarray(_MB)

    out_shapes = (
        jax.ShapeDtypeStruct((n_tok, 1), jnp.int32),
        jax.ShapeDtypeStruct((n_tok, LAT), jnp.float32),
        jax.ShapeDtypeStruct((n_tok, LAT), jnp.float32),
        jax.ShapeDtypeStruct((n_tok, _NC), jnp.float32),
        jax.ShapeDtypeStruct((grid[0], 1, 1), jnp.float32),
    )
    out_specs = (
        pl.BlockSpec((_T, 1), lambda i: (i, 0)),
        pl.BlockSpec((_T, LAT), lambda i: (i, 0)),
        pl.BlockSpec((_T, LAT), lambda i: (i, 0)),
        pl.BlockSpec((_T, _NC), lambda i: (i, 0)),
        pl.BlockSpec((1, 1, 1), lambda i: (i, 0, 0)),
    )
    in_specs = [
        pl.BlockSpec((_T, LAT), lambda i: (i, 0)),
        pl.BlockSpec((LAT, _ND), lambda i: (0, 0)),
        pl.BlockSpec((1, _ND), lambda i: (0, 0)),
        pl.BlockSpec((_ND, LAT), lambda i: (0, 0)),
        pl.BlockSpec((1, LAT), lambda i: (0, 0)),
        pl.BlockSpec((_ND, _NC), lambda i: (0, 0)),
    ]

    tok, zup, zqup, lp, closs_part = pl.pallas_call(
        _lfq_body,
        out_shape=out_shapes,
        grid=grid,
        in_specs=in_specs,
        out_specs=out_specs,
        compiler_params=pltpu.CompilerParams(
            dimension_semantics=("parallel",),
            vmem_limit_bytes=62 * 1024 * 1024,
        ),
        name="lfq_fused",
        interpret=interpret,
    )(x2, Wd, bd2, Wu, bu2, mb)

    tokens = tok.reshape(B, S)
    z_up = zup.reshape(B, S, LAT)
    z_q_up = zqup.reshape(B, S, LAT)
    token_log_probs = lp.reshape(B, S, _NC)
    commit_loss = jnp.sum(closs_part) / (n_tok * _ND)
    return tokens, z_up, z_q_up, token_log_probs, commit_loss


# speedup vs baseline: 14.8339x; 1.0140x over previous
import functools

import jax
import jax.numpy as jnp
import numpy as np
from jax.experimental import pallas as pl
from jax.experimental.pallas import tpu as pltpu

_ND = 12
_NC = 1 << _ND
_LAT = 1024
_T = 1024

_MB = (4.0 * ((np.arange(_NC)[None, :] >> np.arange(_ND)[:, None]) & 1)
       * (2.0 ** np.arange(_ND))[:, None]).astype(np.float32)


def _lfq_body(x_ref, wd_ref, bd_ref, wu_ref, bu_ref, mb_ref,
              tok_ref, zup_ref, zqup_ref, lp_ref, closs_ref):
    x = x_ref[...]
    u = jnp.dot(x, wd_ref[...], preferred_element_type=jnp.float32)
    z = jnp.tanh(u + bd_ref[...])

    bits = z > 0.0
    zq = jnp.where(bits, 1.0, -1.0).astype(jnp.float32)

    zup_ref[...] = (jnp.dot(z, wu_ref[...], preferred_element_type=jnp.float32)
                    + bu_ref[...])
    zqup_ref[...] = (jnp.dot(zq, wu_ref[...], preferred_element_type=jnp.float32)
                     + bu_ref[...])

    ii = jax.lax.broadcasted_iota(jnp.int32, (1, _ND), 1)
    pow2 = (1 << ii).astype(jnp.float32)
    tokf = jnp.sum(jnp.where(bits, pow2, 0.0), axis=1, keepdims=True)
    tok_ref[...] = tokf.astype(jnp.int32)

    i = pl.program_id(0)
    d = z - zq
    part = jnp.sum(jnp.sum(d * d, axis=1, keepdims=True), axis=0,
                   keepdims=True)

    @pl.when(i == 0)
    def _():
        closs_ref[...] = jnp.zeros_like(closs_ref)

    closs_ref[...] += part * (1.0 / (32768 * _ND))

    logits = jnp.dot(z, mb_ref[...], preferred_element_type=jnp.float32)
    c = 4.0 * z * pow2
    sp = jnp.maximum(c, 0.0) + jnp.log1p(jnp.exp(-jnp.abs(c)))
    lse = jnp.sum(sp, axis=1, keepdims=True)
    lp_ref[...] = logits - lse


@functools.partial(jax.jit, static_argnames=("interpret",))
def kernel(x, Wd, bd, Wu, bu, *, interpret=False):
    B, S, LAT = x.shape
    n_tok = B * S
    grid = (n_tok // _T,)

    x2 = x.reshape(n_tok, LAT)
    bd2 = bd.reshape(1, _ND)
    bu2 = bu.reshape(1, LAT)
    mb = jnp.asarray(_MB)

    out_shapes = (
        jax.ShapeDtypeStruct((n_tok, 1), jnp.int32),
        jax.ShapeDtypeStruct((n_tok, LAT), jnp.float32),
        jax.ShapeDtypeStruct((n_tok, LAT), jnp.float32),
        jax.ShapeDtypeStruct((n_tok, _NC), jnp.float32),
        jax.ShapeDtypeStruct((1, 1), jnp.float32),
    )
    out_specs = (
        pl.BlockSpec((_T, 1), lambda i: (i, 0)),
        pl.BlockSpec((_T, LAT), lambda i: (i, 0)),
        pl.BlockSpec((_T, LAT), lambda i: (i, 0)),
        pl.BlockSpec((_T, _NC), lambda i: (i, 0)),
        pl.BlockSpec((1, 1), lambda i: (0, 0)),
    )
    in_specs = [
        pl.BlockSpec((_T, LAT), lambda i: (i, 0)),
        pl.BlockSpec((LAT, _ND), lambda i: (0, 0)),
        pl.BlockSpec((1, _ND), lambda i: (0, 0)),
        pl.BlockSpec((_ND, LAT), lambda i: (0, 0)),
        pl.BlockSpec((1, LAT), lambda i: (0, 0)),
        pl.BlockSpec((_ND, _NC), lambda i: (0, 0)),
    ]

    tok, zup, zqup, lp, closs = pl.pallas_call(
        _lfq_body,
        out_shape=out_shapes,
        grid=grid,
        in_specs=in_specs,
        out_specs=out_specs,
        compiler_params=pltpu.CompilerParams(
            dimension_semantics=("arbitrary",),
            vmem_limit_bytes=62 * 1024 * 1024,
        ),
        name="lfq_fused",
        interpret=interpret,
    )(x2, Wd, bd2, Wu, bu2, mb)

    tokens = tok.reshape(B, S)
    z_up = zup.reshape(B, S, LAT)
    z_q_up = zqup.reshape(B, S, LAT)
    token_log_probs = lp.reshape(B, S, _NC)
    commit_loss = closs.reshape(())
    return tokens, z_up, z_q_up, token_log_probs, commit_loss


# speedup vs baseline: 15.5306x; 1.0470x over previous
import functools

import jax
import jax.numpy as jnp
import numpy as np
from jax.experimental import pallas as pl
from jax.experimental.pallas import tpu as pltpu

_ND = 12
_NC = 1 << _ND
_LAT = 1024
_T = 1024

_MB = (4.0 * ((np.arange(_NC)[None, :] >> np.arange(_ND)[:, None]) & 1)
       * (2.0 ** np.arange(_ND))[:, None]).astype(np.float32)


def _lfq_body(x_ref, wd_ref, bd_ref, wu_ref, bu_ref, mb_ref,
              tok_ref, zup_ref, zqup_ref, lp_ref, closs_ref):
    x = x_ref[...]
    u = jnp.dot(x, wd_ref[...], preferred_element_type=jnp.float32)
    z = jnp.tanh(u + bd_ref[...])

    bits = z > 0.0
    zq = jnp.where(bits, 1.0, -1.0).astype(jnp.float32)

    zup_ref[...] = (jnp.dot(z, wu_ref[...], preferred_element_type=jnp.float32)
                    + bu_ref[...])
    zqup_ref[...] = (jnp.dot(zq, wu_ref[...], preferred_element_type=jnp.float32)
                     + bu_ref[...])

    ii = jax.lax.broadcasted_iota(jnp.int32, (1, _ND), 1)
    pow2 = (1 << ii).astype(jnp.float32)
    bitsf = jnp.where(bits, 1.0, 0.0).astype(jnp.float32)
    tokrow = jax.lax.dot_general(pow2, bitsf, (((1,), (1,)), ((), ())),
                                 preferred_element_type=jnp.float32)
    i = pl.program_id(0)
    nb = 4096 // _T
    tok_ref[pl.ds(i // nb, 1), pl.ds((i % nb) * _T, _T)] = tokrow.astype(jnp.int32)

    d = z - zq
    part = jnp.sum(jnp.sum(d * d, axis=1, keepdims=True), axis=0,
                   keepdims=True)

    @pl.when(i == 0)
    def _():
        closs_ref[...] = jnp.zeros_like(closs_ref)

    closs_ref[...] += part * (1.0 / (32768 * _ND))

    logits = jnp.dot(z, mb_ref[...], preferred_element_type=jnp.float32)
    c = 4.0 * z * pow2
    sp = jnp.maximum(c, 0.0) + jnp.log1p(jnp.exp(-jnp.abs(c)))
    lse = jnp.sum(sp, axis=1, keepdims=True)
    lp_ref[...] = logits - lse


@functools.partial(jax.jit, static_argnames=("interpret",))
def kernel(x, Wd, bd, Wu, bu, *, interpret=False):
    B, S, LAT = x.shape
    n_tok = B * S
    grid = (n_tok // _T,)

    x2 = x.reshape(n_tok, LAT)
    bd2 = bd.reshape(1, _ND)
    bu2 = bu.reshape(1, LAT)
    mb = jnp.asarray(_MB)

    out_shapes = (
        jax.ShapeDtypeStruct((B, S), jnp.int32),
        jax.ShapeDtypeStruct((n_tok, LAT), jnp.float32),
        jax.ShapeDtypeStruct((n_tok, LAT), jnp.float32),
        jax.ShapeDtypeStruct((n_tok, _NC), jnp.float32),
        jax.ShapeDtypeStruct((1, 1), jnp.float32),
    )
    out_specs = (
        pl.BlockSpec((B, S), lambda i: (0, 0)),
        pl.BlockSpec((_T, LAT), lambda i: (i, 0)),
        pl.BlockSpec((_T, LAT), lambda i: (i, 0)),
        pl.BlockSpec((_T, _NC), lambda i: (i, 0)),
        pl.BlockSpec((1, 1), lambda i: (0, 0)),
    )
    in_specs = [
        pl.BlockSpec((_T, LAT), lambda i: (i, 0)),
        pl.BlockSpec((LAT, _ND), lambda i: (0, 0)),
        pl.BlockSpec((1, _ND), lambda i: (0, 0)),
        pl.BlockSpec((_ND, LAT), lambda i: (0, 0)),
        pl.BlockSpec((1, LAT), lambda i: (0, 0)),
        pl.BlockSpec((_ND, _NC), lambda i: (0, 0)),
    ]

    tok, zup, zqup, lp, closs = pl.pallas_call(
        _lfq_body,
        out_shape=out_shapes,
        grid=grid,
        in_specs=in_specs,
        out_specs=out_specs,
        compiler_params=pltpu.CompilerParams(
            dimension_semantics=("arbitrary",),
            vmem_limit_bytes=62 * 1024 * 1024,
        ),
        name="lfq_fused",
        interpret=interpret,
    )(x2, Wd, bd2, Wu, bu2, mb)

    tokens = tok
    z_up = zup.reshape(B, S, LAT)
    z_q_up = zqup.reshape(B, S, LAT)
    token_log_probs = lp.reshape(B, S, _NC)
    commit_loss = closs.reshape(())
    return tokens, z_up, z_q_up, token_log_probs, commit_loss


# speedup vs baseline: 15.5341x; 1.0002x over previous
import functools

import jax
import jax.numpy as jnp
import numpy as np
from jax.experimental import pallas as pl
from jax.experimental.pallas import tpu as pltpu

_ND = 12
_NC = 1 << _ND
_LAT = 1024
_T = 1024

_MB = (4.0 * ((np.arange(_NC)[None, :] >> np.arange(_ND)[:, None]) & 1)
       * (2.0 ** np.arange(_ND))[:, None]).astype(np.float32)


def _lfq_body(x_ref, wd_ref, bd_ref, wu_ref, bu_ref, mb_ref,
              tok_ref, zup_ref, zqup_ref, lp_ref, closs_ref):
    x = x_ref[...]
    u = jnp.dot(x, wd_ref[...], preferred_element_type=jnp.float32)
    z = jnp.tanh(u + bd_ref[...])

    bits = z > 0.0
    zq = jnp.where(bits, 1.0, -1.0).astype(jnp.float32)

    zup_ref[...] = (jnp.dot(z, wu_ref[...], preferred_element_type=jnp.float32)
                    + bu_ref[...])
    zqup_ref[...] = (jnp.dot(zq, wu_ref[...], preferred_element_type=jnp.float32)
                     + bu_ref[...])

    ii = jax.lax.broadcasted_iota(jnp.int32, (1, _ND), 1)
    pow2 = (1 << ii).astype(jnp.float32)
    bitsf = jnp.where(bits, 1.0, 0.0).astype(jnp.float32)
    tokrow = jax.lax.dot_general(pow2, bitsf, (((1,), (1,)), ((), ())),
                                 preferred_element_type=jnp.float32)
    i = pl.program_id(0)
    nb = tok_ref.shape[1] // _T
    tok_ref[pl.ds(i // nb, 1), pl.ds((i % nb) * _T, _T)] = tokrow.astype(jnp.int32)

    d = z - zq
    part = jnp.sum(jnp.sum(d * d, axis=1, keepdims=True), axis=0,
                   keepdims=True)

    @pl.when(i == 0)
    def _():
        closs_ref[...] = jnp.zeros_like(closs_ref)

    closs_ref[...] += part * (1.0 / (pl.num_programs(0) * _T * _ND))

    logits = jnp.dot(z, mb_ref[...], preferred_element_type=jnp.float32)
    c = 4.0 * z * pow2
    sp = jnp.maximum(c, 0.0) + jnp.log1p(jnp.exp(-jnp.abs(c)))
    lse = jnp.sum(sp, axis=1, keepdims=True)
    lp_ref[...] = logits - lse


@functools.partial(jax.jit, static_argnames=("interpret",))
def kernel(x, Wd, bd, Wu, bu, *, interpret=False):
    B, S, LAT = x.shape
    n_tok = B * S
    grid = (n_tok // _T,)

    x2 = x.reshape(n_tok, LAT)
    bd2 = bd.reshape(1, _ND)
    bu2 = bu.reshape(1, LAT)
    mb = jnp.asarray(_MB)

    out_shapes = (
        jax.ShapeDtypeStruct((B, S), jnp.int32),
        jax.ShapeDtypeStruct((n_tok, LAT), jnp.float32),
        jax.ShapeDtypeStruct((n_tok, LAT), jnp.float32),
        jax.ShapeDtypeStruct((n_tok, _NC), jnp.float32),
        jax.ShapeDtypeStruct((1, 1), jnp.float32),
    )
    out_specs = (
        pl.BlockSpec((B, S), lambda i: (0, 0)),
        pl.BlockSpec((_T, LAT), lambda i: (i, 0)),
        pl.BlockSpec((_T, LAT), lambda i: (i, 0)),
        pl.BlockSpec((_T, _NC), lambda i: (i, 0)),
        pl.BlockSpec((1, 1), lambda i: (0, 0)),
    )
    in_specs = [
        pl.BlockSpec((_T, LAT), lambda i: (i, 0)),
        pl.BlockSpec((LAT, _ND), lambda i: (0, 0)),
        pl.BlockSpec((1, _ND), lambda i: (0, 0)),
        pl.BlockSpec((_ND, LAT), lambda i: (0, 0)),
        pl.BlockSpec((1, LAT), lambda i: (0, 0)),
        pl.BlockSpec((_ND, _NC), lambda i: (0, 0)),
    ]

    tok, zup, zqup, lp, closs = pl.pallas_call(
        _lfq_body,
        out_shape=out_shapes,
        grid=grid,
        in_specs=in_specs,
        out_specs=out_specs,
        compiler_params=pltpu.CompilerParams(
            dimension_semantics=("arbitrary",),
            vmem_limit_bytes=62 * 1024 * 1024,
        ),
        name="lfq_fused",
        interpret=interpret,
    )(x2, Wd, bd2, Wu, bu2, mb)

    tokens = tok
    z_up = zup.reshape(B, S, LAT)
    z_q_up = zqup.reshape(B, S, LAT)
    token_log_probs = lp.reshape(B, S, _NC)
    commit_loss = closs.reshape(())
    return tokens, z_up, z_q_up, token_log_probs, commit_loss
